# baseline (speedup 1.0000x reference)
import functools

import jax
import jax.numpy as jnp
from jax import lax
from jax.experimental import pallas as pl
from jax.experimental.pallas import tpu as pltpu
from jax.experimental.pallas import tpu_sc as plsc

N = 10000
E = 160000
D = 256
H = 128
WIDTH = 128
DEGW = 16
N_TILES = 16
EDGES_PER_TILE = E // N_TILES
BATCH = 40
N_BATCH = EDGES_PER_TILE // BATCH
PHASES = 2
PB = N_BATCH // PHASES
NSLOT = 5
ROWS_MAIN = 632
ROWS_LAST = N - 15 * ROWS_MAIN


def _sc_segment_sum(t0, t1, srcs, dsts, zeros, zerosd, onesd):
    mesh = plsc.VectorSubcoreMesh(core_axis_name="c", subcore_axis_name="s")

    @functools.partial(
        pl.kernel,
        mesh=mesh,
        out_type=(
            jax.ShapeDtypeStruct((N, WIDTH), jnp.float32),
            jax.ShapeDtypeStruct((N, WIDTH), jnp.float32),
            jax.ShapeDtypeStruct((N, DEGW), jnp.float32),
            jax.ShapeDtypeStruct((N, DEGW), jnp.float32),
        ),
        scratch_types=[
            pltpu.VMEM((PB, BATCH), jnp.int32),
            pltpu.VMEM((PB, BATCH), jnp.int32),
            pltpu.VMEM((BATCH, DEGW), jnp.float32),
        ] + [pltpu.VMEM((BATCH, WIDTH), jnp.float32) for _ in range(NSLOT)]
        + [pltpu.VMEM_SHARED((N, WIDTH), jnp.float32),
           pltpu.VMEM_SHARED((N, DEGW), jnp.float32)]
        + [pltpu.SemaphoreType.DMA for _ in range(3 * NSLOT)],
        compiler_params=pltpu.CompilerParams(use_tc_tiling_on_sc=False),
    )
    def k(t0_hbm, t1_hbm, srcs_hbm, dsts_hbm, zeros_hbm, zerosd_hbm,
          onesd_hbm, s0_hbm, s1_hbm, sdeg0_hbm, sdeg1_hbm,
          src_v, dst_v, obuf, *rest):
        gbufs = rest[:NSLOT]
        agg_sh = rest[NSLOT]
        deg_sh = rest[NSLOT + 1]
        gsems = rest[NSLOT + 2:NSLOT + 2 + NSLOT]
        ssems = rest[NSLOT + 2 + NSLOT:NSLOT + 2 + 2 * NSLOT]
        dsems = rest[NSLOT + 2 + 2 * NSLOT:]
        c = lax.axis_index("c")
        s = lax.axis_index("s")

        def load_idx(p):
            pltpu.sync_copy(srcs_hbm.at[s, pl.ds(p * PB, PB)], src_v)
            pltpu.sync_copy(dsts_hbm.at[s, pl.ds(p * PB, PB)], dst_v)

        def gather(j, b):
            @pl.when(c == 0)
            def _():
                pltpu.async_copy(t0_hbm.at[src_v.at[j]], gbufs[b], gsems[b])

            @pl.when(c == 1)
            def _():
                pltpu.async_copy(t1_hbm.at[src_v.at[j]], gbufs[b], gsems[b])

        def drain(sem, b):
            pltpu.make_async_copy(t0_hbm.at[pl.ds(0, BATCH)], gbufs[b],
                                  sem).wait()

        def drain_deg(sem):
            pltpu.make_async_copy(onesd_hbm, obuf, sem).wait()

        def scatter(j, b, p):
            pltpu.async_copy(gbufs[b], agg_sh.at[dst_v.at[j]],
                             ssems[b], add=True)

            @pl.when(c == p)
            def _():
                pltpu.async_copy(obuf, deg_sh.at[dst_v.at[j]],
                                 dsems[b], add=True)

        def drain_prev(b, p):
            drain(ssems[(b - 1) % NSLOT], (b - 1) % NSLOT)

            @pl.when(c == p)
            def _():
                drain_deg(dsems[(b - 1) % NSLOT])

        load_idx(0)
        for b in range(NSLOT - 1):
            gather(b, b)
        pltpu.sync_copy(onesd_hbm, obuf)

        @pl.when(s < 15)
        def _():
            rows = pl.ds(s * ROWS_MAIN, ROWS_MAIN)
            pltpu.sync_copy(zeros_hbm, agg_sh.at[rows])
            pltpu.sync_copy(zerosd_hbm, deg_sh.at[rows])

        @pl.when(s == 15)
        def _():
            rows = pl.ds(15 * ROWS_MAIN, ROWS_LAST)
            pltpu.sync_copy(zeros_hbm.at[pl.ds(0, ROWS_LAST)],
                            agg_sh.at[rows])
            pltpu.sync_copy(zerosd_hbm.at[pl.ds(0, ROWS_LAST)],
                            deg_sh.at[rows])

        plsc.subcore_barrier()

        for p in range(PHASES):
            if p > 0:
                load_idx(p)
                for b in range(NSLOT - 1):
                    gather(b, b)

            def outer(j0, carry, p=p):
                for b in range(NSLOT):
                    j = j0 + b
                    drain(gsems[b], b)
                    @pl.when(j > 0)
                    def _():
                        drain_prev(b, p)

                    @pl.when(j + NSLOT - 1 < PB)
                    def _():
                        gather(j + NSLOT - 1, (b - 1) % NSLOT)

                    scatter(j, b, p)
                return carry

            lax.fori_loop(0, PB // NSLOT,
                          lambda g, cc: outer(g * NSLOT, cc), 0)
            bt = (PB - 1) % NSLOT
            drain(ssems[bt], bt)

            @pl.when(c == p)
            def _():
                drain_deg(dsems[bt])

        plsc.subcore_barrier()

        def writeback(src_sh, dst_hbm):
            @pl.when(s < 15)
            def _():
                rows = pl.ds(s * ROWS_MAIN, ROWS_MAIN)
                pltpu.sync_copy(src_sh.at[rows], dst_hbm.at[rows])

            @pl.when(s == 15)
            def _():
                rows = pl.ds(15 * ROWS_MAIN, ROWS_LAST)
                pltpu.sync_copy(src_sh.at[rows], dst_hbm.at[rows])

        @pl.when(c == 0)
        def _():
            writeback(agg_sh, s0_hbm)
            writeback(deg_sh, sdeg0_hbm)

        @pl.when(c == 1)
        def _():
            writeback(agg_sh, s1_hbm)
            writeback(deg_sh, sdeg1_hbm)

    return k(t0, t1, srcs, dsts, zeros, zerosd, onesd)


def _tc_finish(s0, s1, sdeg0, sdeg1, x, w, gamma, beta):
    def body(s0_ref, s1_ref, d0_ref, d1_ref, x_ref, w_ref, g_ref, b_ref,
             o_ref):
        deg = jnp.maximum(d0_ref[:, :1] + d1_ref[:, :1], 1.0)
        ssum = jnp.concatenate([s0_ref[...], s1_ref[...]], axis=1)
        m = ssum / deg
        rst = lax.dot_general(m, w_ref[...], (((1,), (0,)), ((), ())),
                              preferred_element_type=jnp.float32)
        rst = rst + x_ref[...]
        mu = jnp.mean(rst, axis=0, keepdims=True)
        var = jnp.mean((rst - mu) ** 2, axis=0, keepdims=True)
        h = (rst - mu) * lax.rsqrt(var + 1e-5) * g_ref[...] + b_ref[...]
        o_ref[...] = jnp.maximum(h, 0.0)

    return pl.pallas_call(
        body,
        out_shape=jax.ShapeDtypeStruct((N, D), jnp.float32),
    )(s0, s1, sdeg0, sdeg1, x, w, gamma.reshape(1, D), beta.reshape(1, D))


def kernel(x, edge_index, W_src, gamma, beta):
    src = edge_index[0].astype(jnp.int32)
    dst = edge_index[1].astype(jnp.int32)
    t0 = x[:, :H]
    t1 = x[:, H:]
    srcs = src.reshape(N_TILES, N_BATCH, BATCH)
    dsts = dst.reshape(N_TILES, N_BATCH, BATCH)
    zeros = jnp.zeros((ROWS_MAIN, WIDTH), jnp.float32)
    zerosd = jnp.zeros((ROWS_MAIN, DEGW), jnp.float32)
    onesd = jnp.ones((BATCH, DEGW), jnp.float32)
    s0, s1, sdeg0, sdeg1 = _sc_segment_sum(t0, t1, srcs, dsts, zeros,
                                           zerosd, onesd)
    return _tc_finish(s0, s1, sdeg0, sdeg1, x, W_src, gamma, beta)


# speedup vs baseline: 1.0175x; 1.0175x over previous
import functools

import jax
import jax.numpy as jnp
from jax import lax
from jax.experimental import pallas as pl
from jax.experimental.pallas import tpu as pltpu
from jax.experimental.pallas import tpu_sc as plsc

N = 10000
E = 160000
D = 256
H = 128
WIDTH = 128
DEGW = 16
N_TILES = 16
EDGES_PER_TILE = E // N_TILES
BATCH = 40
N_BATCH = EDGES_PER_TILE // BATCH
PHASES = 2
PB = N_BATCH // PHASES
NSLOT = 5
ROWS_MAIN = 632
ROWS_LAST = N - 15 * ROWS_MAIN


def _sc_segment_sum(t0, t1, srcs, dsts, zeros, zerosd, onesd):
    mesh = plsc.VectorSubcoreMesh(core_axis_name="c", subcore_axis_name="s")

    @functools.partial(
        pl.kernel,
        mesh=mesh,
        out_type=(
            jax.ShapeDtypeStruct((N, WIDTH), jnp.float32),
            jax.ShapeDtypeStruct((N, WIDTH), jnp.float32),
            jax.ShapeDtypeStruct((N, DEGW), jnp.float32),
        ),
        scratch_types=[
            pltpu.VMEM((PB, BATCH), jnp.int32),
            pltpu.VMEM((PB, BATCH), jnp.int32),
            pltpu.VMEM((BATCH, DEGW), jnp.float32),
        ] + [pltpu.VMEM((BATCH, WIDTH), jnp.float32) for _ in range(NSLOT)]
        + [pltpu.VMEM_SHARED((N, WIDTH), jnp.float32),
           pltpu.VMEM_SHARED((N, DEGW), jnp.float32)]
        + [pltpu.SemaphoreType.DMA for _ in range(3 * NSLOT)],
        compiler_params=pltpu.CompilerParams(use_tc_tiling_on_sc=False),
    )
    def k(t0_hbm, t1_hbm, srcs_hbm, dsts_hbm, zeros_hbm, zerosd_hbm,
          onesd_hbm, s0_hbm, s1_hbm, sdeg_hbm, src_v, dst_v, obuf, *rest):
        gbufs = rest[:NSLOT]
        agg_sh = rest[NSLOT]
        deg_sh = rest[NSLOT + 1]
        gsems = rest[NSLOT + 2:NSLOT + 2 + NSLOT]
        ssems = rest[NSLOT + 2 + NSLOT:NSLOT + 2 + 2 * NSLOT]
        dsems = rest[NSLOT + 2 + 2 * NSLOT:]
        c = lax.axis_index("c")
        s = lax.axis_index("s")

        def load_idx(p):
            pltpu.sync_copy(srcs_hbm.at[s, pl.ds(p * PB, PB)], src_v)
            pltpu.sync_copy(dsts_hbm.at[s, pl.ds(p * PB, PB)], dst_v)

        def gather(j, b):
            @pl.when(c == 0)
            def _():
                pltpu.async_copy(t0_hbm.at[src_v.at[j]], gbufs[b], gsems[b])

            @pl.when(c == 1)
            def _():
                pltpu.async_copy(t1_hbm.at[src_v.at[j]], gbufs[b], gsems[b])

        def drain(sem, b):
            pltpu.make_async_copy(t0_hbm.at[pl.ds(0, BATCH)], gbufs[b],
                                  sem).wait()

        def drain_deg(sem):
            pltpu.make_async_copy(onesd_hbm, obuf, sem).wait()

        def scatter(j, b):
            pltpu.async_copy(gbufs[b], agg_sh.at[dst_v.at[j]],
                             ssems[b], add=True)

            @pl.when(c == 0)
            def _():
                pltpu.async_copy(obuf, deg_sh.at[dst_v.at[j]],
                                 dsems[b], add=True)

        def drain_prev(b):
            drain(ssems[(b - 1) % NSLOT], (b - 1) % NSLOT)

            @pl.when(c == 0)
            def _():
                drain_deg(dsems[(b - 1) % NSLOT])

        load_idx(0)
        for b in range(NSLOT - 1):
            gather(b, b)
        pltpu.sync_copy(onesd_hbm, obuf)

        @pl.when(s < 15)
        def _():
            rows = pl.ds(s * ROWS_MAIN, ROWS_MAIN)
            pltpu.sync_copy(zeros_hbm, agg_sh.at[rows])
            pltpu.sync_copy(zerosd_hbm, deg_sh.at[rows])

        @pl.when(s == 15)
        def _():
            rows = pl.ds(15 * ROWS_MAIN, ROWS_LAST)
            pltpu.sync_copy(zeros_hbm.at[pl.ds(0, ROWS_LAST)],
                            agg_sh.at[rows])
            pltpu.sync_copy(zerosd_hbm.at[pl.ds(0, ROWS_LAST)],
                            deg_sh.at[rows])

        plsc.subcore_barrier()

        for p in range(PHASES):
            if p > 0:
                load_idx(p)
                for b in range(NSLOT - 1):
                    gather(b, b)

            def outer(j0, carry):
                for b in range(NSLOT):
                    j = j0 + b
                    drain(gsems[b], b)
                    @pl.when(j > 0)
                    def _():
                        drain_prev(b)

                    @pl.when(j + NSLOT - 1 < PB)
                    def _():
                        gather(j + NSLOT - 1, (b - 1) % NSLOT)

                    scatter(j, b)
                return carry

            lax.fori_loop(0, PB // NSLOT,
                          lambda g, cc: outer(g * NSLOT, cc), 0)
            bt = (PB - 1) % NSLOT
            drain(ssems[bt], bt)

            @pl.when(c == 0)
            def _():
                drain_deg(dsems[bt])

        plsc.subcore_barrier()

        def writeback(src_sh, dst_hbm):
            @pl.when(s < 15)
            def _():
                rows = pl.ds(s * ROWS_MAIN, ROWS_MAIN)
                pltpu.sync_copy(src_sh.at[rows], dst_hbm.at[rows])

            @pl.when(s == 15)
            def _():
                rows = pl.ds(15 * ROWS_MAIN, ROWS_LAST)
                pltpu.sync_copy(src_sh.at[rows], dst_hbm.at[rows])

        @pl.when(c == 0)
        def _():
            writeback(agg_sh, s0_hbm)
            writeback(deg_sh, sdeg_hbm)

        @pl.when(c == 1)
        def _():
            writeback(agg_sh, s1_hbm)

    return k(t0, t1, srcs, dsts, zeros, zerosd, onesd)


def _tc_finish(s0, s1, sdeg, x, w, gamma, beta):
    def body(s0_ref, s1_ref, d_ref, x_ref, w_ref, g_ref, b_ref, o_ref):
        inv = 1.0 / jnp.maximum(d_ref[:, :1], 1.0)
        dims = (((1,), (0,)), ((), ()))
        rst = lax.dot_general(s0_ref[...] * inv, w_ref[:H], dims,
                              preferred_element_type=jnp.float32)
        rst = rst + lax.dot_general(s1_ref[...] * inv, w_ref[H:], dims,
                                    preferred_element_type=jnp.float32)
        rst = rst + x_ref[...]
        mu = jnp.mean(rst, axis=0, keepdims=True)
        var = jnp.mean((rst - mu) ** 2, axis=0, keepdims=True)
        h = (rst - mu) * lax.rsqrt(var + 1e-5) * g_ref[...] + b_ref[...]
        o_ref[...] = jnp.maximum(h, 0.0)

    return pl.pallas_call(
        body,
        out_shape=jax.ShapeDtypeStruct((N, D), jnp.float32),
    )(s0, s1, sdeg, x, w, gamma.reshape(1, D), beta.reshape(1, D))


def kernel(x, edge_index, W_src, gamma, beta):
    src = edge_index[0].astype(jnp.int32)
    dst = edge_index[1].astype(jnp.int32)
    t0 = x[:, :H]
    t1 = x[:, H:]
    srcs = src.reshape(N_TILES, N_BATCH, BATCH)
    dsts = dst.reshape(N_TILES, N_BATCH, BATCH)
    zeros = jnp.zeros((ROWS_MAIN, WIDTH), jnp.float32)
    zerosd = jnp.zeros((ROWS_MAIN, DEGW), jnp.float32)
    onesd = jnp.ones((BATCH, DEGW), jnp.float32)
    s0, s1, sdeg = _sc_segment_sum(t0, t1, srcs, dsts, zeros, zerosd, onesd)
    return _tc_finish(s0, s1, sdeg, x, W_src, gamma, beta)
